# Initial kernel scaffold; baseline (speedup 1.0000x reference)
import functools

import jax
import jax.numpy as jnp
from jax import lax
from jax.experimental import pallas as pl
from jax.experimental.pallas import tpu as pltpu
from jax.experimental.pallas import tpu_sc as plsc

N = 10000
E = 320000
D_IN = 128
H = 64

NW = 32
EPT = E // NW
EB = 125
NJ = EPT // EB
RPS = N // 16
DEGW = 16

_mesh = plsc.VectorSubcoreMesh(core_axis_name="c", subcore_axis_name="s")


def _fill_rows(buf, width, val):
    def body(i, _):
        for k in range(width // 16):
            buf[i, pl.ds(k * 16, 16)] = jnp.full((16,), val, jnp.float32)
        return 0
    lax.fori_loop(0, EB, body, 0)


@functools.partial(
    pl.kernel,
    out_type=jax.ShapeDtypeStruct((2, N, DEGW), jnp.float32),
    mesh=_mesh,
    scratch_types=[
        pltpu.VMEM_SHARED((N, DEGW), jnp.float32),
        pltpu.VMEM((NJ, EB), jnp.int32),
        pltpu.VMEM((EB, DEGW), jnp.float32),
    ],
)
def _sc_degree(dst_hbm, out_hbm, acc, idx_d, buf):
    cid = lax.axis_index("c")
    sid = lax.axis_index("s")
    wid = sid * 2 + cid

    _fill_rows(buf, DEGW, 0.0)
    for k in range(RPS // EB):
        pltpu.sync_copy(buf, acc.at[pl.ds(sid * RPS + k * EB, EB)])
    plsc.subcore_barrier()

    pltpu.sync_copy(dst_hbm.at[wid], idx_d)
    _fill_rows(buf, DEGW, 1.0)

    def body(j, _):
        pltpu.sync_copy(buf, acc.at[idx_d.at[j]], add=True)
        return 0
    lax.fori_loop(0, NJ, body, 0)
    plsc.subcore_barrier()

    pltpu.sync_copy(acc.at[pl.ds(sid * RPS, RPS)],
                    out_hbm.at[cid, pl.ds(sid * RPS, RPS)])


@functools.partial(
    pl.kernel,
    out_type=jax.ShapeDtypeStruct((2, N, H), jnp.float32),
    mesh=_mesh,
    scratch_types=[
        pltpu.VMEM_SHARED((N, H), jnp.float32),
        pltpu.VMEM((NJ, EB), jnp.int32),
        pltpu.VMEM((NJ, EB), jnp.int32),
        pltpu.VMEM((EB, H), jnp.float32),
        pltpu.VMEM((EB, H), jnp.float32),
        pltpu.SemaphoreType.DMA,
    ],
)
def _sc_aggregate(g_hbm, src_hbm, dst_hbm, out_hbm,
                  acc, idx_s, idx_d, rows, zbuf, sem):
    cid = lax.axis_index("c")
    sid = lax.axis_index("s")
    wid = sid * 2 + cid

    _fill_rows(zbuf, H, 0.0)
    for k in range(RPS // EB):
        pltpu.sync_copy(zbuf, acc.at[pl.ds(sid * RPS + k * EB, EB)])
    plsc.subcore_barrier()

    pltpu.sync_copy(src_hbm.at[wid], idx_s)
    pltpu.sync_copy(dst_hbm.at[wid], idx_d)

    def body(j, _):
        pltpu.async_copy(g_hbm.at[idx_s.at[j]], rows, sem).wait()
        pltpu.sync_copy(rows, acc.at[idx_d.at[j]], add=True)
        return 0
    lax.fori_loop(0, NJ, body, 0)
    plsc.subcore_barrier()

    pltpu.sync_copy(acc.at[pl.ds(sid * RPS, RPS)],
                    out_hbm.at[cid, pl.ds(sid * RPS, RPS)])


_RB = 1000
_GRID = N // _RB


def _tc1_body(x_ref, dp_ref, w1_ref, g1_ref, dinv_ref):
    t = dp_ref[0] + dp_ref[1]
    deg = jnp.sum(t, axis=1, keepdims=True) + 1.0
    dinv = lax.rsqrt(deg)
    u = jnp.dot(x_ref[...], w1_ref[...], preferred_element_type=jnp.float32)
    g1_ref[...] = u * dinv
    dinv_ref[...] = dinv


def _tc2_body(p_ref, g_ref, dinv_ref, w_ref, b_ref, g2_ref):
    dinv = dinv_ref[...]
    h = dinv * (p_ref[0] + p_ref[1] + g_ref[...]) + b_ref[...]
    g2_ref[...] = jnp.dot(h, w_ref[...], preferred_element_type=jnp.float32) * dinv


def _tc3_body(p_ref, g_ref, dinv_ref, b2_ref,
              l1w_ref, l1b_ref, l2w_ref, l2b_ref, l3w_ref, l3b_ref, out_ref):
    dinv = dinv_ref[...]
    h = dinv * (p_ref[0] + p_ref[1] + g_ref[...]) + b2_ref[...]
    a = jnp.maximum(
        jnp.dot(h, l1w_ref[...], preferred_element_type=jnp.float32)
        + l1b_ref[...], 0.0)
    a = jnp.maximum(
        jnp.dot(a, l2w_ref[...], preferred_element_type=jnp.float32)
        + l2b_ref[...], 0.0)
    out_ref[...] = (jnp.dot(a, l3w_ref[...], preferred_element_type=jnp.float32)
                    + l3b_ref[...])


def _row_spec(w):
    return pl.BlockSpec((_RB, w), lambda i: (i, 0))


def _par_spec(w):
    return pl.BlockSpec((2, _RB, w), lambda i: (0, i, 0))


def _full_spec(shape):
    return pl.BlockSpec(shape, lambda i: tuple(0 for _ in shape))


def kernel(x, edge_index, W1, b1, W2, b2,
           lin1_W, lin1_b, lin2_W, lin2_b, lin3_W, lin3_b):
    src = edge_index[0].astype(jnp.int32).reshape(NW, NJ, EB)
    dst = edge_index[1].astype(jnp.int32).reshape(NW, NJ, EB)

    deg_par = _sc_degree(dst)

    g1, dinv = pl.pallas_call(
        _tc1_body,
        grid=(_GRID,),
        in_specs=[_row_spec(D_IN), _par_spec(DEGW), _full_spec((D_IN, H))],
        out_specs=[_row_spec(H), _row_spec(1)],
        out_shape=[jax.ShapeDtypeStruct((N, H), jnp.float32),
                   jax.ShapeDtypeStruct((N, 1), jnp.float32)],
    )(x, deg_par, W1)

    p1 = _sc_aggregate(g1, src, dst)

    g2 = pl.pallas_call(
        _tc2_body,
        grid=(_GRID,),
        in_specs=[_par_spec(H), _row_spec(H), _row_spec(1),
                  _full_spec((H, H)), _full_spec((1, H))],
        out_specs=_row_spec(H),
        out_shape=jax.ShapeDtypeStruct((N, H), jnp.float32),
    )(p1, g1, dinv, W2, b1.reshape(1, H))

    p2 = _sc_aggregate(g2, src, dst)

    out = pl.pallas_call(
        _tc3_body,
        grid=(_GRID,),
        in_specs=[_par_spec(H), _row_spec(H), _row_spec(1),
                  _full_spec((1, H)),
                  _full_spec((H, 64)), _full_spec((1, 64)),
                  _full_spec((64, 32)), _full_spec((1, 32)),
                  _full_spec((32, 1)), _full_spec((1, 1))],
        out_specs=_row_spec(1),
        out_shape=jax.ShapeDtypeStruct((N, 1), jnp.float32),
    )(p2, g2, dinv, b2.reshape(1, H),
      lin1_W, lin1_b.reshape(1, 64),
      lin2_W, lin2_b.reshape(1, 32),
      lin3_W, lin3_b.reshape(1, 1))

    return out


# speedup vs baseline: 27.8438x; 27.8438x over previous
import functools

import jax
import jax.numpy as jnp
from jax import lax
from jax.experimental import pallas as pl
from jax.experimental.pallas import tpu as pltpu
from jax.experimental.pallas import tpu_sc as plsc

N = 10000
E = 320000
D_IN = 128
H = 64

NW = 32
EPT = E // NW
EB = 125
NJ = EPT // EB
RPS = N // 16
DEGW = 16

_mesh = plsc.VectorSubcoreMesh(core_axis_name="c", subcore_axis_name="s")


def _fill_rows(buf, width, val):
    def body(i, _):
        for k in range(width // 16):
            buf[i, pl.ds(k * 16, 16)] = jnp.full((16,), val, jnp.float32)
        return 0
    lax.fori_loop(0, EB, body, 0)


@functools.partial(
    pl.kernel,
    out_type=jax.ShapeDtypeStruct((2, N, DEGW), jnp.float32),
    mesh=_mesh,
    compiler_params=pltpu.CompilerParams(use_tc_tiling_on_sc=False),
    scratch_types=[
        pltpu.VMEM_SHARED((N, DEGW), jnp.float32),
        pltpu.VMEM((NJ, EB), jnp.int32),
        pltpu.VMEM((EB, DEGW), jnp.float32),
    ],
)
def _sc_degree(dst_hbm, out_hbm, acc, idx_d, buf):
    cid = lax.axis_index("c")
    sid = lax.axis_index("s")
    wid = sid * 2 + cid

    _fill_rows(buf, DEGW, 0.0)
    for k in range(RPS // EB):
        pltpu.sync_copy(buf, acc.at[pl.ds(sid * RPS + k * EB, EB)])
    plsc.subcore_barrier()

    pltpu.sync_copy(dst_hbm.at[wid], idx_d)
    _fill_rows(buf, DEGW, 1.0)

    def body(j, _):
        pltpu.sync_copy(buf, acc.at[idx_d.at[j]], add=True)
        return 0
    lax.fori_loop(0, NJ, body, 0)
    plsc.subcore_barrier()

    pltpu.sync_copy(acc.at[pl.ds(sid * RPS, RPS)],
                    out_hbm.at[cid, pl.ds(sid * RPS, RPS)])


@functools.partial(
    pl.kernel,
    out_type=jax.ShapeDtypeStruct((2, N, H), jnp.float32),
    mesh=_mesh,
    compiler_params=pltpu.CompilerParams(use_tc_tiling_on_sc=False),
    scratch_types=[
        pltpu.VMEM_SHARED((N, H), jnp.float32),
        pltpu.VMEM((NJ, EB), jnp.int32),
        pltpu.VMEM((NJ, EB), jnp.int32),
        pltpu.VMEM((EB, H), jnp.float32),
        pltpu.VMEM((EB, H), jnp.float32),
        pltpu.SemaphoreType.DMA,
    ],
)
def _sc_aggregate(g_hbm, src_hbm, dst_hbm, out_hbm,
                  acc, idx_s, idx_d, rows, zbuf, sem):
    cid = lax.axis_index("c")
    sid = lax.axis_index("s")
    wid = sid * 2 + cid

    _fill_rows(zbuf, H, 0.0)
    for k in range(RPS // EB):
        pltpu.sync_copy(zbuf, acc.at[pl.ds(sid * RPS + k * EB, EB)])
    plsc.subcore_barrier()

    pltpu.sync_copy(src_hbm.at[wid], idx_s)
    pltpu.sync_copy(dst_hbm.at[wid], idx_d)

    def body(j, _):
        pltpu.async_copy(g_hbm.at[idx_s.at[j]], rows, sem).wait()
        pltpu.sync_copy(rows, acc.at[idx_d.at[j]], add=True)
        return 0
    lax.fori_loop(0, NJ, body, 0)
    plsc.subcore_barrier()

    pltpu.sync_copy(acc.at[pl.ds(sid * RPS, RPS)],
                    out_hbm.at[cid, pl.ds(sid * RPS, RPS)])


_RB = 1000
_GRID = N // _RB


def _tc1_body(x_ref, dp_ref, w1_ref, g1_ref, dinv_ref):
    t = dp_ref[0] + dp_ref[1]
    deg = jnp.sum(t, axis=1, keepdims=True) * (1.0 / DEGW) + 1.0
    dinv = lax.rsqrt(deg)
    u = jnp.dot(x_ref[...], w1_ref[...], preferred_element_type=jnp.float32)
    g1_ref[...] = u * dinv
    dinv_ref[...] = dinv


def _tc2_body(p_ref, g_ref, dinv_ref, w_ref, b_ref, g2_ref):
    dinv = dinv_ref[...]
    h = dinv * (p_ref[0] + p_ref[1] + g_ref[...]) + b_ref[...]
    g2_ref[...] = jnp.dot(h, w_ref[...], preferred_element_type=jnp.float32) * dinv


def _tc3_body(p_ref, g_ref, dinv_ref, b2_ref,
              l1w_ref, l1b_ref, l2w_ref, l2b_ref, l3w_ref, l3b_ref, out_ref):
    dinv = dinv_ref[...]
    h = dinv * (p_ref[0] + p_ref[1] + g_ref[...]) + b2_ref[...]
    a = jnp.maximum(
        jnp.dot(h, l1w_ref[...], preferred_element_type=jnp.float32)
        + l1b_ref[...], 0.0)
    a = jnp.maximum(
        jnp.dot(a, l2w_ref[...], preferred_element_type=jnp.float32)
        + l2b_ref[...], 0.0)
    out_ref[...] = (jnp.dot(a, l3w_ref[...], preferred_element_type=jnp.float32)
                    + l3b_ref[...])


def _row_spec(w):
    return pl.BlockSpec((_RB, w), lambda i: (i, 0))


def _par_spec(w):
    return pl.BlockSpec((2, _RB, w), lambda i: (0, i, 0))


def _full_spec(shape):
    return pl.BlockSpec(shape, lambda i: tuple(0 for _ in shape))


def kernel(x, edge_index, W1, b1, W2, b2,
           lin1_W, lin1_b, lin2_W, lin2_b, lin3_W, lin3_b):
    src = edge_index[0].astype(jnp.int32).reshape(NW, NJ, EB)
    dst = edge_index[1].astype(jnp.int32).reshape(NW, NJ, EB)

    deg_par = _sc_degree(dst)

    g1, dinv = pl.pallas_call(
        _tc1_body,
        grid=(_GRID,),
        in_specs=[_row_spec(D_IN), _par_spec(DEGW), _full_spec((D_IN, H))],
        out_specs=[_row_spec(H), _row_spec(1)],
        out_shape=[jax.ShapeDtypeStruct((N, H), jnp.float32),
                   jax.ShapeDtypeStruct((N, 1), jnp.float32)],
    )(x, deg_par, W1)

    p1 = _sc_aggregate(g1, src, dst)

    g2 = pl.pallas_call(
        _tc2_body,
        grid=(_GRID,),
        in_specs=[_par_spec(H), _row_spec(H), _row_spec(1),
                  _full_spec((H, H)), _full_spec((1, H))],
        out_specs=_row_spec(H),
        out_shape=jax.ShapeDtypeStruct((N, H), jnp.float32),
    )(p1, g1, dinv, W2, b1.reshape(1, H))

    p2 = _sc_aggregate(g2, src, dst)

    out = pl.pallas_call(
        _tc3_body,
        grid=(_GRID,),
        in_specs=[_par_spec(H), _row_spec(H), _row_spec(1),
                  _full_spec((1, H)),
                  _full_spec((H, 64)), _full_spec((1, 64)),
                  _full_spec((64, 32)), _full_spec((1, 32)),
                  _full_spec((32, 1)), _full_spec((1, 1))],
        out_specs=_row_spec(1),
        out_shape=jax.ShapeDtypeStruct((N, 1), jnp.float32),
    )(p2, g2, dinv, b2.reshape(1, H),
      lin1_W, lin1_b.reshape(1, 64),
      lin2_W, lin2_b.reshape(1, 32),
      lin3_W, lin3_b.reshape(1, 1))

    return out


# speedup vs baseline: 32.6152x; 1.1714x over previous
import functools

import jax
import jax.numpy as jnp
from jax import lax
from jax.experimental import pallas as pl
from jax.experimental.pallas import tpu as pltpu
from jax.experimental.pallas import tpu_sc as plsc

N = 10000
E = 320000
D_IN = 128
H = 64

NW = 32
EPT = E // NW
EB = 125
NJ = EPT // EB
RPS = N // 16
DEGW = 16

_mesh = plsc.VectorSubcoreMesh(core_axis_name="c", subcore_axis_name="s")


def _fill_rows(buf, width, val):
    def body(i, _):
        for k in range(width // 16):
            buf[i, pl.ds(k * 16, 16)] = jnp.full((16,), val, jnp.float32)
        return 0
    lax.fori_loop(0, EB, body, 0)


@functools.partial(
    pl.kernel,
    out_type=jax.ShapeDtypeStruct((2, N, DEGW), jnp.float32),
    mesh=_mesh,
    compiler_params=pltpu.CompilerParams(use_tc_tiling_on_sc=False),
    scratch_types=[
        pltpu.VMEM_SHARED((N, DEGW), jnp.float32),
        pltpu.VMEM((NJ, EB), jnp.int32),
        pltpu.VMEM((EB, DEGW), jnp.float32),
    ],
)
def _sc_degree(dst_hbm, out_hbm, acc, idx_d, buf):
    cid = lax.axis_index("c")
    sid = lax.axis_index("s")
    wid = sid * 2 + cid

    _fill_rows(buf, DEGW, 0.0)
    for k in range(RPS // EB):
        pltpu.sync_copy(buf, acc.at[pl.ds(sid * RPS + k * EB, EB)])
    plsc.subcore_barrier()

    pltpu.sync_copy(dst_hbm.at[wid], idx_d)
    _fill_rows(buf, DEGW, 1.0)

    def body(j, _):
        pltpu.sync_copy(buf, acc.at[idx_d.at[j]], add=True)
        return 0
    lax.fori_loop(0, NJ, body, 0)
    plsc.subcore_barrier()

    pltpu.sync_copy(acc.at[pl.ds(sid * RPS, RPS)],
                    out_hbm.at[cid, pl.ds(sid * RPS, RPS)])


@functools.partial(
    pl.kernel,
    out_type=jax.ShapeDtypeStruct((2, N, H), jnp.float32),
    mesh=_mesh,
    compiler_params=pltpu.CompilerParams(use_tc_tiling_on_sc=False),
    scratch_types=[
        pltpu.VMEM_SHARED((N, H), jnp.float32),
        pltpu.VMEM((NJ, EB), jnp.int32),
        pltpu.VMEM((NJ, EB), jnp.int32),
        pltpu.VMEM((2, EB, H), jnp.float32),
        pltpu.VMEM((EB, H), jnp.float32),
        pltpu.SemaphoreType.DMA((2,)),
    ],
)
def _sc_aggregate(g_hbm, src_hbm, dst_hbm, out_hbm,
                  acc, idx_s, idx_d, rows, zbuf, sem):
    cid = lax.axis_index("c")
    sid = lax.axis_index("s")
    wid = sid * 2 + cid

    _fill_rows(zbuf, H, 0.0)
    for k in range(RPS // EB):
        pltpu.sync_copy(zbuf, acc.at[pl.ds(sid * RPS + k * EB, EB)])
    plsc.subcore_barrier()

    pltpu.sync_copy(src_hbm.at[wid], idx_s)
    pltpu.sync_copy(dst_hbm.at[wid], idx_d)

    pltpu.async_copy(g_hbm.at[idx_s.at[0]], rows.at[0], sem.at[0])

    def body(i, _):
        for b in range(2):
            j = i * 2 + b
            pltpu.make_async_copy(
                g_hbm.at[idx_s.at[0]], rows.at[b], sem.at[b]).wait()
            jn = jnp.minimum(j + 1, NJ - 1)
            pltpu.async_copy(
                g_hbm.at[idx_s.at[jn]], rows.at[1 - b], sem.at[1 - b])
            pltpu.sync_copy(rows.at[b], acc.at[idx_d.at[j]], add=True)
        return 0
    lax.fori_loop(0, NJ // 2, body, 0)
    pltpu.make_async_copy(g_hbm.at[idx_s.at[0]], rows.at[0], sem.at[0]).wait()
    plsc.subcore_barrier()

    pltpu.sync_copy(acc.at[pl.ds(sid * RPS, RPS)],
                    out_hbm.at[cid, pl.ds(sid * RPS, RPS)])


_RB = 1000
_GRID = N // _RB


def _tc1_body(x_ref, dp_ref, w1_ref, g1_ref, dinv_ref):
    t = dp_ref[0] + dp_ref[1]
    deg = jnp.sum(t, axis=1, keepdims=True) * (1.0 / DEGW) + 1.0
    dinv = lax.rsqrt(deg)
    u = jnp.dot(x_ref[...], w1_ref[...], preferred_element_type=jnp.float32)
    g1_ref[...] = u * dinv
    dinv_ref[...] = dinv


def _tc2_body(p_ref, g_ref, dinv_ref, w_ref, b_ref, g2_ref):
    dinv = dinv_ref[...]
    h = dinv * (p_ref[0] + p_ref[1] + g_ref[...]) + b_ref[...]
    g2_ref[...] = jnp.dot(h, w_ref[...], preferred_element_type=jnp.float32) * dinv


def _tc3_body(p_ref, g_ref, dinv_ref, b2_ref,
              l1w_ref, l1b_ref, l2w_ref, l2b_ref, l3w_ref, l3b_ref, out_ref):
    dinv = dinv_ref[...]
    h = dinv * (p_ref[0] + p_ref[1] + g_ref[...]) + b2_ref[...]
    a = jnp.maximum(
        jnp.dot(h, l1w_ref[...], preferred_element_type=jnp.float32)
        + l1b_ref[...], 0.0)
    a = jnp.maximum(
        jnp.dot(a, l2w_ref[...], preferred_element_type=jnp.float32)
        + l2b_ref[...], 0.0)
    out_ref[...] = (jnp.dot(a, l3w_ref[...], preferred_element_type=jnp.float32)
                    + l3b_ref[...])


def _row_spec(w):
    return pl.BlockSpec((_RB, w), lambda i: (i, 0))


def _par_spec(w):
    return pl.BlockSpec((2, _RB, w), lambda i: (0, i, 0))


def _full_spec(shape):
    return pl.BlockSpec(shape, lambda i: tuple(0 for _ in shape))


def kernel(x, edge_index, W1, b1, W2, b2,
           lin1_W, lin1_b, lin2_W, lin2_b, lin3_W, lin3_b):
    src = edge_index[0].astype(jnp.int32).reshape(NW, NJ, EB)
    dst = edge_index[1].astype(jnp.int32).reshape(NW, NJ, EB)

    deg_par = _sc_degree(dst)

    g1, dinv = pl.pallas_call(
        _tc1_body,
        grid=(_GRID,),
        in_specs=[_row_spec(D_IN), _par_spec(DEGW), _full_spec((D_IN, H))],
        out_specs=[_row_spec(H), _row_spec(1)],
        out_shape=[jax.ShapeDtypeStruct((N, H), jnp.float32),
                   jax.ShapeDtypeStruct((N, 1), jnp.float32)],
    )(x, deg_par, W1)

    p1 = _sc_aggregate(g1, src, dst)

    g2 = pl.pallas_call(
        _tc2_body,
        grid=(_GRID,),
        in_specs=[_par_spec(H), _row_spec(H), _row_spec(1),
                  _full_spec((H, H)), _full_spec((1, H))],
        out_specs=_row_spec(H),
        out_shape=jax.ShapeDtypeStruct((N, H), jnp.float32),
    )(p1, g1, dinv, W2, b1.reshape(1, H))

    p2 = _sc_aggregate(g2, src, dst)

    out = pl.pallas_call(
        _tc3_body,
        grid=(_GRID,),
        in_specs=[_par_spec(H), _row_spec(H), _row_spec(1),
                  _full_spec((1, H)),
                  _full_spec((H, 64)), _full_spec((1, 64)),
                  _full_spec((64, 32)), _full_spec((1, 32)),
                  _full_spec((32, 1)), _full_spec((1, 1))],
        out_specs=_row_spec(1),
        out_shape=jax.ShapeDtypeStruct((N, 1), jnp.float32),
    )(p2, g2, dinv, b2.reshape(1, H),
      lin1_W, lin1_b.reshape(1, 64),
      lin2_W, lin2_b.reshape(1, 32),
      lin3_W, lin3_b.reshape(1, 1))

    return out


# speedup vs baseline: 43.4151x; 1.3311x over previous
import functools

import jax
import jax.numpy as jnp
from jax import lax
from jax.experimental import pallas as pl
from jax.experimental.pallas import tpu as pltpu
from jax.experimental.pallas import tpu_sc as plsc

N = 10000
E = 320000
D_IN = 128
H = 64

NW = 32
EPT = E // NW
EB = 125
NJ = EPT // EB
RPS = N // 16
DEGW = 16

_mesh = plsc.VectorSubcoreMesh(core_axis_name="c", subcore_axis_name="s")


def _fill_rows(buf, width, val):
    def body(i, _):
        for k in range(width // 16):
            buf[i, pl.ds(k * 16, 16)] = jnp.full((16,), val, jnp.float32)
        return 0
    lax.fori_loop(0, EB, body, 0)


@functools.partial(
    pl.kernel,
    out_type=jax.ShapeDtypeStruct((2, N, DEGW), jnp.float32),
    mesh=_mesh,
    compiler_params=pltpu.CompilerParams(use_tc_tiling_on_sc=False),
    scratch_types=[
        pltpu.VMEM_SHARED((N, DEGW), jnp.float32),
        pltpu.VMEM((NJ, EB), jnp.int32),
        pltpu.VMEM((EB, DEGW), jnp.float32),
    ],
)
def _sc_degree(dst_hbm, out_hbm, acc, idx_d, buf):
    cid = lax.axis_index("c")
    sid = lax.axis_index("s")
    wid = sid * 2 + cid

    _fill_rows(buf, DEGW, 0.0)
    for k in range(RPS // EB):
        pltpu.sync_copy(buf, acc.at[pl.ds(sid * RPS + k * EB, EB)])
    plsc.subcore_barrier()

    pltpu.sync_copy(dst_hbm.at[wid], idx_d)
    _fill_rows(buf, DEGW, 1.0)

    def body(j, _):
        pltpu.sync_copy(buf, acc.at[idx_d.at[j]], add=True)
        return 0
    lax.fori_loop(0, NJ, body, 0)
    plsc.subcore_barrier()

    pltpu.sync_copy(acc.at[pl.ds(sid * RPS, RPS)],
                    out_hbm.at[cid, pl.ds(sid * RPS, RPS)])


@functools.partial(
    pl.kernel,
    out_type=jax.ShapeDtypeStruct((2, N, H), jnp.float32),
    mesh=_mesh,
    compiler_params=pltpu.CompilerParams(use_tc_tiling_on_sc=False),
    scratch_types=[
        pltpu.VMEM_SHARED((N, H), jnp.float32),
        pltpu.VMEM((NJ, EB), jnp.int32),
        pltpu.VMEM((NJ, EB), jnp.int32),
        pltpu.VMEM((4, EB, H), jnp.float32),
        pltpu.VMEM((EB, H), jnp.float32),
        pltpu.SemaphoreType.DMA((4,)),
    ],
)
def _sc_aggregate(g_hbm, src_hbm, dst_hbm, out_hbm,
                  acc, idx_s, idx_d, rows, zbuf, sem):
    cid = lax.axis_index("c")
    sid = lax.axis_index("s")
    wid = sid * 2 + cid

    _fill_rows(zbuf, H, 0.0)
    for k in range(RPS // EB):
        pltpu.sync_copy(zbuf, acc.at[pl.ds(sid * RPS + k * EB, EB)])
    plsc.subcore_barrier()

    pltpu.sync_copy(src_hbm.at[wid], idx_s)
    pltpu.sync_copy(dst_hbm.at[wid], idx_d)

    for b in range(3):
        pltpu.async_copy(g_hbm.at[idx_s.at[b]], rows.at[b], sem.at[b])

    def body(i, _):
        for b in range(4):
            j = i * 4 + b
            pltpu.make_async_copy(
                g_hbm.at[idx_s.at[0]], rows.at[b], sem.at[b]).wait()
            jn = jnp.minimum(j + 3, NJ - 1)
            bn = (b + 3) % 4
            pltpu.async_copy(g_hbm.at[idx_s.at[jn]], rows.at[bn], sem.at[bn])
            pltpu.sync_copy(rows.at[b], acc.at[idx_d.at[j]], add=True)
        return 0
    lax.fori_loop(0, NJ // 4, body, 0)
    for b in range(3):
        pltpu.make_async_copy(
            g_hbm.at[idx_s.at[0]], rows.at[b], sem.at[b]).wait()
    plsc.subcore_barrier()

    pltpu.sync_copy(acc.at[pl.ds(sid * RPS, RPS)],
                    out_hbm.at[cid, pl.ds(sid * RPS, RPS)])


_RB = 1000
_GRID = N // _RB


def _tc1_body(x_ref, dp_ref, w1_ref, g1_ref, dinv_ref):
    t = dp_ref[0] + dp_ref[1]
    deg = jnp.sum(t, axis=1, keepdims=True) * (1.0 / DEGW) + 1.0
    dinv = lax.rsqrt(deg)
    u = jnp.dot(x_ref[...], w1_ref[...], preferred_element_type=jnp.float32)
    g1_ref[...] = u * dinv
    dinv_ref[...] = dinv


def _tc2_body(p_ref, g_ref, dinv_ref, w_ref, b_ref, g2_ref):
    dinv = dinv_ref[...]
    h = dinv * (p_ref[0] + p_ref[1] + g_ref[...]) + b_ref[...]
    g2_ref[...] = jnp.dot(h, w_ref[...], preferred_element_type=jnp.float32) * dinv


def _tc3_body(p_ref, g_ref, dinv_ref, b2_ref,
              l1w_ref, l1b_ref, l2w_ref, l2b_ref, l3w_ref, l3b_ref, out_ref):
    dinv = dinv_ref[...]
    h = dinv * (p_ref[0] + p_ref[1] + g_ref[...]) + b2_ref[...]
    a = jnp.maximum(
        jnp.dot(h, l1w_ref[...], preferred_element_type=jnp.float32)
        + l1b_ref[...], 0.0)
    a = jnp.maximum(
        jnp.dot(a, l2w_ref[...], preferred_element_type=jnp.float32)
        + l2b_ref[...], 0.0)
    out_ref[...] = (jnp.dot(a, l3w_ref[...], preferred_element_type=jnp.float32)
                    + l3b_ref[...])


def _row_spec(w):
    return pl.BlockSpec((_RB, w), lambda i: (i, 0))


def _par_spec(w):
    return pl.BlockSpec((2, _RB, w), lambda i: (0, i, 0))


def _full_spec(shape):
    return pl.BlockSpec(shape, lambda i: tuple(0 for _ in shape))


def kernel(x, edge_index, W1, b1, W2, b2,
           lin1_W, lin1_b, lin2_W, lin2_b, lin3_W, lin3_b):
    src = edge_index[0].astype(jnp.int32).reshape(NW, NJ, EB)
    dst = edge_index[1].astype(jnp.int32).reshape(NW, NJ, EB)

    deg_par = _sc_degree(dst)

    g1, dinv = pl.pallas_call(
        _tc1_body,
        grid=(_GRID,),
        in_specs=[_row_spec(D_IN), _par_spec(DEGW), _full_spec((D_IN, H))],
        out_specs=[_row_spec(H), _row_spec(1)],
        out_shape=[jax.ShapeDtypeStruct((N, H), jnp.float32),
                   jax.ShapeDtypeStruct((N, 1), jnp.float32)],
    )(x, deg_par, W1)

    p1 = _sc_aggregate(g1, src, dst)

    g2 = pl.pallas_call(
        _tc2_body,
        grid=(_GRID,),
        in_specs=[_par_spec(H), _row_spec(H), _row_spec(1),
                  _full_spec((H, H)), _full_spec((1, H))],
        out_specs=_row_spec(H),
        out_shape=jax.ShapeDtypeStruct((N, H), jnp.float32),
    )(p1, g1, dinv, W2, b1.reshape(1, H))

    p2 = _sc_aggregate(g2, src, dst)

    out = pl.pallas_call(
        _tc3_body,
        grid=(_GRID,),
        in_specs=[_par_spec(H), _row_spec(H), _row_spec(1),
                  _full_spec((1, H)),
                  _full_spec((H, 64)), _full_spec((1, 64)),
                  _full_spec((64, 32)), _full_spec((1, 32)),
                  _full_spec((32, 1)), _full_spec((1, 1))],
        out_specs=_row_spec(1),
        out_shape=jax.ShapeDtypeStruct((N, 1), jnp.float32),
    )(p2, g2, dinv, b2.reshape(1, H),
      lin1_W, lin1_b.reshape(1, 64),
      lin2_W, lin2_b.reshape(1, 32),
      lin3_W, lin3_b.reshape(1, 1))

    return out
